# baseline (speedup 1.0000x reference)
import jax
import jax.numpy as jnp
from jax.experimental import pallas as pl
from jax.experimental.pallas import tpu as pltpu

_CHUNK = 4096
_SLOTS = 4


def _make_mdma_kernel(n_chunks, chunk, rows_packed, lanes):
    def body(x_hbm, w_ref, b_ref, o_hbm, x_buf, o_buf, in_sems, out_sems):
        xv = x_hbm
        ov = o_hbm

        def start_in(slot, t):
            pltpu.make_async_copy(
                xv.at[pl.ds(t * chunk, chunk)], x_buf.at[slot],
                in_sems.at[slot]).start()

        def wait_in(slot):
            pltpu.make_async_copy(
                xv.at[pl.ds(0, chunk)], x_buf.at[slot],
                in_sems.at[slot]).wait()

        def start_out(slot, t):
            pltpu.make_async_copy(
                o_buf.at[slot], ov.at[pl.ds(t * chunk, chunk)],
                out_sems.at[slot]).start()

        def wait_out(slot):
            pltpu.make_async_copy(
                o_buf.at[slot], ov.at[pl.ds(0, chunk)],
                out_sems.at[slot]).wait()

        for s in range(min(_SLOTS, n_chunks)):
            start_in(s, s)

        def step(t, carry):
            slot = jax.lax.rem(t, _SLOTS)
            wait_in(slot)

            @pl.when(t >= _SLOTS)
            def _():
                wait_out(slot)

            xb = x_buf[slot].astype(jnp.bfloat16)
            z = jnp.dot(xb, w_ref[...], preferred_element_type=jnp.float32)
            o_buf[slot] = jnp.maximum(z + b_ref[...], 0.0)
            start_out(slot, t)

            @pl.when(t + _SLOTS < n_chunks)
            def _():
                start_in(slot, t + _SLOTS)

            return carry

        jax.lax.fori_loop(0, n_chunks, step, 0)

        for s in range(min(_SLOTS, n_chunks)):
            wait_out(s)

    return body


def _emitter_fallback(x, w_eff, b_row, n, d, e):
    tile = min(8192, ((n + 7) // 8) * 8)
    n_pad = ((n + tile - 1) // tile) * tile
    xp = jnp.pad(x, ((0, n_pad - n), (0, 0))) if n_pad != n else x
    out = pl.pallas_call(
        lambda x_ref, w_ref, b_ref, o_ref: o_ref.__setitem__(
            ..., jnp.maximum(
                jnp.dot(x_ref[...], w_ref[...],
                        preferred_element_type=jnp.float32) + b_ref[...],
                0.0)),
        out_shape=jax.ShapeDtypeStruct((n_pad, e), jnp.float32),
        grid=(n_pad // tile,),
        in_specs=[
            pl.BlockSpec((tile, d), lambda i: (i, 0)),
            pl.BlockSpec((d, e), lambda i: (0, 0)),
            pl.BlockSpec((1, e), lambda i: (0, 0)),
        ],
        out_specs=pl.BlockSpec((tile, e), lambda i: (i, 0)),
        compiler_params=pltpu.CompilerParams(
            dimension_semantics=("parallel",)),
    )(xp, w_eff, b_row)
    return out[:n]


def kernel(x, beta, w, b):
    n, d = x.shape
    e = w.shape[1]
    w_eff = beta * w
    b_row = b.reshape(1, e)

    p = 128 // d if (d < 128 and 128 % d == 0 and d == e) else 1
    rows_packed = n // p if (p > 1 and n % p == 0) else 0
    n_chunks = rows_packed // _CHUNK if rows_packed else 0
    if p == 1 or not n_chunks or rows_packed % _CHUNK or n_chunks < _SLOTS:
        return _emitter_fallback(x, w_eff, b_row, n, d, e)

    w_p = jax.scipy.linalg.block_diag(*([w_eff] * p)).astype(jnp.bfloat16)
    b_p = jnp.tile(b_row, (1, p))
    lanes = p * d
    xp = x.reshape(rows_packed, lanes)

    out = pl.pallas_call(
        _make_mdma_kernel(n_chunks, _CHUNK, rows_packed, lanes),
        out_shape=jax.ShapeDtypeStruct((rows_packed, lanes), jnp.float32),
        in_specs=[
            pl.BlockSpec(memory_space=pl.ANY),
            pl.BlockSpec(memory_space=pltpu.VMEM),
            pl.BlockSpec(memory_space=pltpu.VMEM),
        ],
        out_specs=pl.BlockSpec(memory_space=pl.ANY),
        scratch_shapes=[
            pltpu.VMEM((_SLOTS, _CHUNK, lanes), jnp.float32),
            pltpu.VMEM((_SLOTS, _CHUNK, lanes), jnp.float32),
            pltpu.SemaphoreType.DMA((_SLOTS,)),
            pltpu.SemaphoreType.DMA((_SLOTS,)),
        ],
    )(xp, w_p, b_p)
    return out.reshape(n, e)


# speedup vs baseline: 1.0329x; 1.0329x over previous
import jax
import jax.numpy as jnp
from jax.experimental import pallas as pl
from jax.experimental.pallas import tpu as pltpu

_CHUNK = 16384
_SLOTS = 4


def _make_kernel(n_chunks, chunk, p):
    cq = chunk // p

    def body(x_hbm, w_ref, b_ref, o_hbm, x_buf, o_buf, in_sems, out_sems):
        def start_in(slot, t):
            pltpu.make_async_copy(
                x_hbm.at[pl.ds(t * chunk, chunk)], x_buf.at[slot],
                in_sems.at[slot]).start()

        def wait_in(slot):
            pltpu.make_async_copy(
                x_hbm.at[pl.ds(0, chunk)], x_buf.at[slot],
                in_sems.at[slot]).wait()

        def start_out(slot, t):
            pltpu.make_async_copy(
                o_buf.at[slot], o_hbm.at[pl.ds(t * cq, cq)],
                out_sems.at[slot]).start()

        def wait_out(slot):
            pltpu.make_async_copy(
                o_buf.at[slot], o_hbm.at[pl.ds(0, cq)],
                out_sems.at[slot]).wait()

        for s in range(min(_SLOTS, n_chunks)):
            start_in(s, s)

        def step(t, carry):
            slot = jax.lax.rem(t, _SLOTS)
            wait_in(slot)

            @pl.when(t >= _SLOTS)
            def _():
                wait_out(slot)

            xc = x_buf[slot]
            xq = jnp.concatenate(
                [xc[k * cq:(k + 1) * cq, :] for k in range(p)],
                axis=1).astype(jnp.bfloat16)
            z = jnp.dot(xq, w_ref[...], preferred_element_type=jnp.float32)
            o_buf[slot] = jnp.maximum(z + b_ref[...], 0.0)
            start_out(slot, t)

            @pl.when(t + _SLOTS < n_chunks)
            def _():
                start_in(slot, t + _SLOTS)

            return carry

        jax.lax.fori_loop(0, n_chunks, step, 0)

        for s in range(min(_SLOTS, n_chunks)):
            wait_out(s)

    return body


def _emitter_fallback(x, w_eff, b_row, n, d, e):
    tile = min(8192, ((n + 7) // 8) * 8)
    n_pad = ((n + tile - 1) // tile) * tile
    xp = jnp.pad(x, ((0, n_pad - n), (0, 0))) if n_pad != n else x
    out = pl.pallas_call(
        lambda x_ref, w_ref, b_ref, o_ref: o_ref.__setitem__(
            ..., jnp.maximum(
                jnp.dot(x_ref[...], w_ref[...],
                        preferred_element_type=jnp.float32) + b_ref[...],
                0.0)),
        out_shape=jax.ShapeDtypeStruct((n_pad, e), jnp.float32),
        grid=(n_pad // tile,),
        in_specs=[
            pl.BlockSpec((tile, d), lambda i: (i, 0)),
            pl.BlockSpec((d, e), lambda i: (0, 0)),
            pl.BlockSpec((1, e), lambda i: (0, 0)),
        ],
        out_specs=pl.BlockSpec((tile, e), lambda i: (i, 0)),
        compiler_params=pltpu.CompilerParams(
            dimension_semantics=("parallel",)),
    )(xp, w_eff, b_row)
    return out[:n]


def kernel(x, beta, w, b):
    n, d = x.shape
    e = w.shape[1]
    w_eff = beta * w
    b_row = b.reshape(1, e)

    p = 128 // d if (d < 128 and 128 % d == 0 and d == e) else 1
    ok = (p > 1 and n % _CHUNK == 0 and _CHUNK % p == 0
          and n // _CHUNK >= _SLOTS)
    if not ok:
        return _emitter_fallback(x, w_eff, b_row, n, d, e)

    n_chunks = n // _CHUNK
    cq = _CHUNK // p
    lanes = p * d
    w_p = jax.scipy.linalg.block_diag(*([w_eff] * p)).astype(jnp.bfloat16)
    b_p = jnp.tile(b_row, (1, p))

    out = pl.pallas_call(
        _make_kernel(n_chunks, _CHUNK, p),
        out_shape=jax.ShapeDtypeStruct((n // p, lanes), jnp.float32),
        in_specs=[
            pl.BlockSpec(memory_space=pl.ANY),
            pl.BlockSpec(memory_space=pltpu.VMEM),
            pl.BlockSpec(memory_space=pltpu.VMEM),
        ],
        out_specs=pl.BlockSpec(memory_space=pl.ANY),
        scratch_shapes=[
            pltpu.VMEM((_SLOTS, _CHUNK, d), jnp.float32),
            pltpu.VMEM((_SLOTS, cq, lanes), jnp.float32),
            pltpu.SemaphoreType.DMA((_SLOTS,)),
            pltpu.SemaphoreType.DMA((_SLOTS,)),
        ],
    )(x, w_p, b_p)
    return (out.reshape(n_chunks, cq, p, e)
            .transpose(0, 2, 1, 3)
            .reshape(n, e))


# speedup vs baseline: 1.2104x; 1.1718x over previous
import jax
import jax.numpy as jnp
from jax.experimental import pallas as pl
from jax.experimental.pallas import tpu as pltpu

_TILE = 16384


def _fused_kernel(x_ref, w_ref, b_ref, out_ref):
    z = jnp.dot(x_ref[...], w_ref[...], preferred_element_type=jnp.float32)
    out_ref[...] = jnp.maximum(z + b_ref[...], 0.0)


def kernel(x, beta, w, b):
    n, d = x.shape
    e = w.shape[1]
    w_eff = beta * w
    b_row = b.reshape(1, e)

    tile = min(_TILE, ((n + 7) // 8) * 8)
    n_pad = ((n + tile - 1) // tile) * tile
    if n_pad != n:
        x = jnp.pad(x, ((0, n_pad - n), (0, 0)))

    out = pl.pallas_call(
        _fused_kernel,
        out_shape=jax.ShapeDtypeStruct((n_pad, e), jnp.float32),
        grid=(n_pad // tile,),
        in_specs=[
            pl.BlockSpec((tile, d), lambda i: (i, 0)),
            pl.BlockSpec((d, e), lambda i: (0, 0)),
            pl.BlockSpec((1, e), lambda i: (0, 0)),
        ],
        out_specs=pl.BlockSpec((tile, e), lambda i: (i, 0)),
        compiler_params=pltpu.CompilerParams(
            dimension_semantics=("parallel",)),
    )(x, w_eff, b_row)
    return out[:n]
